# baseline (device time: 15463 ns/iter reference)
import jax
import jax.numpy as jnp
from jax import lax
from jax.experimental import pallas as pl
from jax.experimental.pallas import tpu as pltpu

N_DEV = 4
B, SQ, SKV, HQ_LOCAL, DH = 2, 128, 128, 4, 64
D_MODEL = 512
BLK = 64
BF = jnp.bfloat16


def _body(idx_ref, x_ref, wq_ref, k_ref, v_ref, wo_ref, out_ref,
          comm_ref, send_sems, recv_sems):
    del idx_ref
    my = lax.axis_index("i")
    p1 = my ^ 1
    p2 = 3 - my

    barrier = pltpu.get_barrier_semaphore()
    for nbr in (p1, p2):
        pl.semaphore_signal(barrier, inc=1, device_id=(nbr,),
                            device_id_type=pl.DeviceIdType.MESH)

    qb = lax.broadcasted_iota(jnp.int32, (SQ, SKV), 0) // BLK
    kb = lax.broadcasted_iota(jnp.int32, (SQ, SKV), 1) // BLK
    mask = (qb == kb) | (kb == 0) | ((qb + kb) % 3 == 0)

    wqb = wq_ref[...].astype(BF)
    wob = wo_ref[...].astype(BF)

    def partial_for_batch(b):
        q = jnp.dot(x_ref[b].astype(BF), wqb,
                    preferred_element_type=jnp.float32)
        qbf = (q * 0.125).astype(BF)
        acc = None
        for h in range(HQ_LOCAL):
            qs = qbf[:, h * DH:(h + 1) * DH]
            ks = k_ref[b, :, h * DH:(h + 1) * DH].astype(BF)
            vs = v_ref[b, :, h * DH:(h + 1) * DH].astype(BF)
            s = lax.dot_general(qs, ks, (((1,), (1,)), ((), ())),
                                preferred_element_type=jnp.float32)
            w = jnp.where(mask, jnp.exp(s), 0.0)
            denom = jnp.sum(w, axis=-1, keepdims=True)
            ctx = jnp.dot(w.astype(BF), vs,
                          preferred_element_type=jnp.float32) / denom
            part = jnp.dot(ctx.astype(BF), wob[h * DH:(h + 1) * DH, :],
                           preferred_element_type=jnp.float32)
            acc = part if acc is None else acc + part
        return acc

    def xchg(send_slot, recv_slot, sem, partner):
        return pltpu.make_async_remote_copy(
            src_ref=comm_ref.at[send_slot],
            dst_ref=comm_ref.at[recv_slot],
            send_sem=send_sems.at[sem],
            recv_sem=recv_sems.at[sem],
            device_id=(partner,),
            device_id_type=pl.DeviceIdType.MESH,
        )

    part0 = partial_for_batch(0)
    comm_ref[0, :, :] = part0.astype(BF)
    pl.semaphore_wait(barrier, 2)
    r10 = xchg(0, 2, 0, p1)
    r10.start()

    part1 = partial_for_batch(1)
    comm_ref[1, :, :] = part1.astype(BF)
    r11 = xchg(1, 3, 1, p1)
    r11.start()

    r10.wait()
    s1_0 = part0 + comm_ref[2, :, :].astype(jnp.float32)
    comm_ref[4, :, :] = s1_0.astype(BF)
    r20 = xchg(4, 6, 2, p2)
    r20.start()

    r11.wait()
    s1_1 = part1 + comm_ref[3, :, :].astype(jnp.float32)
    comm_ref[5, :, :] = s1_1.astype(BF)
    r21 = xchg(5, 7, 3, p2)
    r21.start()

    r20.wait()
    out_ref[0] = s1_0 + comm_ref[6, :, :].astype(jnp.float32)
    r21.wait()
    out_ref[1] = s1_1 + comm_ref[7, :, :].astype(jnp.float32)


def kernel(x, Wq, K_ext, V_ext, Wo):
    my = lax.axis_index("i")
    idx = jnp.reshape(my, (1,)).astype(jnp.int32)
    k3 = K_ext.reshape(B, SKV, 16 * DH)
    v3 = V_ext.reshape(B, SKV, 16 * DH)

    grid_spec = pltpu.PrefetchScalarGridSpec(
        num_scalar_prefetch=1,
        grid=(1,),
        in_specs=[
            pl.BlockSpec((B, SQ, D_MODEL), lambda i, s: (0, 0, 0)),
            pl.BlockSpec((D_MODEL, HQ_LOCAL * DH), lambda i, s: (0, 0)),
            pl.BlockSpec((B, SKV, HQ_LOCAL * DH),
                         lambda i, s: (0, 0, s[0])),
            pl.BlockSpec((B, SKV, HQ_LOCAL * DH),
                         lambda i, s: (0, 0, s[0])),
            pl.BlockSpec((HQ_LOCAL * DH, D_MODEL), lambda i, s: (0, 0)),
        ],
        out_specs=pl.BlockSpec((B, SQ, D_MODEL), lambda i, s: (0, 0, 0)),
        scratch_shapes=[
            pltpu.VMEM((8, SQ, D_MODEL), BF),
            pltpu.SemaphoreType.DMA((4,)),
            pltpu.SemaphoreType.DMA((4,)),
        ],
    )

    return pl.pallas_call(
        _body,
        out_shape=jax.ShapeDtypeStruct((B, SQ, D_MODEL), jnp.float32),
        grid_spec=grid_spec,
        compiler_params=pltpu.CompilerParams(collective_id=0),
    )(idx, x, Wq, k3, v3, Wo)


# device time: 14343 ns/iter; 1.0781x vs baseline; 1.0781x over previous
import jax
import jax.numpy as jnp
from jax import lax
from jax.experimental import pallas as pl
from jax.experimental.pallas import tpu as pltpu

N_DEV = 4
B, SQ, SKV, HQ_LOCAL, DH = 2, 128, 128, 4, 64
D_MODEL = 512
BLK = 64
BF = jnp.bfloat16


def _body(x_ref, wq_ref, k_ref, v_ref, wo_ref, out_ref,
          comm_ref, send_sems, recv_sems):
    my = lax.axis_index("i")
    p1 = my ^ 1
    p2 = 3 - my

    barrier = pltpu.get_barrier_semaphore()
    for nbr in (p1, p2):
        pl.semaphore_signal(barrier, inc=1, device_id=(nbr,),
                            device_id_type=pl.DeviceIdType.MESH)

    qb = lax.broadcasted_iota(jnp.int32, (SQ, SKV), 0) // BLK
    kb = lax.broadcasted_iota(jnp.int32, (SQ, SKV), 1) // BLK
    mask = (qb == kb) | (kb == 0) | ((qb + kb) % 3 == 0)

    wqb = wq_ref[...].astype(BF)
    wob = wo_ref[...].astype(BF)

    def partial_for_batch(b):
        q = jnp.dot(x_ref[b].astype(BF), wqb,
                    preferred_element_type=jnp.float32)
        qbf = (q * 0.125).astype(BF)
        acc = None
        for h in range(HQ_LOCAL):
            qs = qbf[:, h * DH:(h + 1) * DH]
            ks = k_ref[b, :, h, :].astype(BF)
            vs = v_ref[b, :, h, :].astype(BF)
            s = lax.dot_general(qs, ks, (((1,), (1,)), ((), ())),
                                preferred_element_type=jnp.float32)
            w = jnp.where(mask, jnp.exp(s), 0.0)
            denom = jnp.sum(w, axis=-1, keepdims=True)
            ctx = jnp.dot(w.astype(BF), vs,
                          preferred_element_type=jnp.float32) / denom
            part = jnp.dot(ctx.astype(BF), wob[h * DH:(h + 1) * DH, :],
                           preferred_element_type=jnp.float32)
            acc = part if acc is None else acc + part
        return acc

    def xchg(send_slot, recv_slot, sem, partner):
        return pltpu.make_async_remote_copy(
            src_ref=comm_ref.at[send_slot],
            dst_ref=comm_ref.at[recv_slot],
            send_sem=send_sems.at[sem],
            recv_sem=recv_sems.at[sem],
            device_id=(partner,),
            device_id_type=pl.DeviceIdType.MESH,
        )

    HC = SQ // 2
    parts = [None] * 4
    step1 = [None] * 4
    step2 = [None] * 4

    part0 = partial_for_batch(0)
    parts[0] = part0[0:HC, :]
    parts[1] = part0[HC:SQ, :]
    comm_ref[0, :, :] = parts[0].astype(BF)
    comm_ref[1, :, :] = parts[1].astype(BF)
    pl.semaphore_wait(barrier, 2)
    for c in (0, 1):
        step1[c] = xchg(c, 4 + c, c, p1)
        step1[c].start()

    part1 = partial_for_batch(1)
    parts[2] = part1[0:HC, :]
    parts[3] = part1[HC:SQ, :]
    comm_ref[2, :, :] = parts[2].astype(BF)
    comm_ref[3, :, :] = parts[3].astype(BF)
    for c in (2, 3):
        step1[c] = xchg(c, 4 + c, c, p1)
        step1[c].start()

    s1 = [None] * 4
    for c in range(4):
        step1[c].wait()
        s1[c] = parts[c] + comm_ref[4 + c, :, :].astype(jnp.float32)
        comm_ref[8 + c, :, :] = s1[c].astype(BF)
        step2[c] = xchg(8 + c, 12 + c, 4 + c, p2)
        step2[c].start()

    for c in range(4):
        step2[c].wait()
        b, r = c // 2, (c % 2) * HC
        out_ref[b, r:r + HC, :] = \
            s1[c] + comm_ref[12 + c, :, :].astype(jnp.float32)


def kernel(x, Wq, K_ext, V_ext, Wo):
    my = lax.axis_index("i")
    K = lax.dynamic_slice_in_dim(K_ext, my * HQ_LOCAL, HQ_LOCAL, axis=2)
    V = lax.dynamic_slice_in_dim(V_ext, my * HQ_LOCAL, HQ_LOCAL, axis=2)

    return pl.pallas_call(
        _body,
        out_shape=jax.ShapeDtypeStruct((B, SQ, D_MODEL), jnp.float32),
        in_specs=[pl.BlockSpec(memory_space=pltpu.VMEM)] * 5,
        out_specs=pl.BlockSpec(memory_space=pltpu.VMEM),
        scratch_shapes=[
            pltpu.VMEM((16, SQ // 2, D_MODEL), BF),
            pltpu.SemaphoreType.DMA((8,)),
            pltpu.SemaphoreType.DMA((8,)),
        ],
        compiler_params=pltpu.CompilerParams(collective_id=0),
    )(x, Wq, K, V, Wo)
